# baseline (speedup 1.0000x reference)
import functools

import jax
import jax.numpy as jnp
from jax import lax
from jax.experimental import pallas as pl
from jax.experimental.pallas import tpu as pltpu
from jax.experimental.pallas import tpu_sc as plsc

D_MODEL = 64
D_PAD = 128
B_TOTAL = 16384 * 20
NUM_WORKERS = 32
CHUNK = 128
K = 8
ROWS_PER_W = B_TOTAL // (NUM_WORKERS * CHUNK)
NUM_BLOCKS = ROWS_PER_W // K

_mesh = plsc.VectorSubcoreMesh(core_axis_name="c", subcore_axis_name="s")


@functools.partial(
    pl.kernel,
    mesh=_mesh,
    out_type=jax.ShapeDtypeStruct((B_TOTAL, D_PAD), jnp.float32),
    scratch_types=[
        pltpu.VMEM((K, CHUNK), jnp.int32),
        pltpu.VMEM((6, CHUNK, D_PAD), jnp.float32),
        pltpu.SemaphoreType.DMA((6,)),
        pltpu.SemaphoreType.DMA((6,)),
    ],
)
def _gather_kernel(idx_hbm, table_hbm, out_hbm, idx_v, rows_v, sem_g, sem_s):
    wid = lax.axis_index("s") * 2 + lax.axis_index("c")
    base_row = wid * ROWS_PER_W

    def _gather(h, buf, row):
        return pltpu.async_copy(table_hbm.at[idx_v.at[h]], rows_v.at[buf],
                                sem_g.at[buf])

    def _store(h, buf, row):
        return pltpu.async_copy(
            rows_v.at[buf],
            out_hbm.at[pl.ds((row + h) * CHUNK, CHUNK)], sem_s.at[buf])

    def body(blk, carry):
        row = base_row + blk * K
        pltpu.sync_copy(idx_hbm.at[pl.ds(row, K)], idx_v)
        g = [None] * K
        st = [None] * K
        for h in range(4):
            g[h] = _gather(h, h % 6, row)
        for h in range(K):
            g[h].wait()
            if h + 4 < K:
                if h >= 2:
                    st[h - 2].wait()
                g[h + 4] = _gather(h + 4, (h + 4) % 6, row)
            st[h] = _store(h, h % 6, row)
        st[K - 2].wait()
        st[K - 1].wait()
        return carry

    lax.fori_loop(0, NUM_BLOCKS, body, 0)


def kernel(token_ids, weight):
    idx = token_ids.reshape(B_TOTAL // CHUNK, CHUNK).astype(jnp.int32)
    wp = jnp.pad(weight, ((0, 0), (0, D_PAD - D_MODEL)))
    out = _gather_kernel(idx, wp)
    return out[:, :D_MODEL].reshape(token_ids.shape + (D_MODEL,))


# speedup vs baseline: 1.0019x; 1.0019x over previous
import functools

import jax
import jax.numpy as jnp
from jax import lax
from jax.experimental import pallas as pl
from jax.experimental.pallas import tpu as pltpu
from jax.experimental.pallas import tpu_sc as plsc

D_MODEL = 64
D_PAD = 128
B_TOTAL = 16384 * 20
NUM_WORKERS = 32
CHUNK = 128
K = 16
ROWS_PER_W = B_TOTAL // (NUM_WORKERS * CHUNK)
NUM_BLOCKS = ROWS_PER_W // K

_mesh = plsc.VectorSubcoreMesh(core_axis_name="c", subcore_axis_name="s")


@functools.partial(
    pl.kernel,
    mesh=_mesh,
    out_type=jax.ShapeDtypeStruct((B_TOTAL, D_PAD), jnp.float32),
    scratch_types=[
        pltpu.VMEM((K, CHUNK), jnp.int32),
        pltpu.VMEM((6, CHUNK, D_PAD), jnp.float32),
        pltpu.SemaphoreType.DMA((6,)),
        pltpu.SemaphoreType.DMA((6,)),
    ],
)
def _gather_kernel(idx_hbm, table_hbm, out_hbm, idx_v, rows_v, sem_g, sem_s):
    wid = lax.axis_index("s") * 2 + lax.axis_index("c")
    base_row = wid * ROWS_PER_W

    def _gather(h, buf, row):
        return pltpu.async_copy(table_hbm.at[idx_v.at[h]], rows_v.at[buf],
                                sem_g.at[buf])

    def _store(h, buf, row):
        return pltpu.async_copy(
            rows_v.at[buf],
            out_hbm.at[pl.ds((row + h) * CHUNK, CHUNK)], sem_s.at[buf])

    def body(blk, carry):
        row = base_row + blk * K
        pltpu.sync_copy(idx_hbm.at[pl.ds(row, K)], idx_v)
        g = [None] * K
        st = [None] * K
        for h in range(4):
            g[h] = _gather(h, h % 6, row)
        for h in range(K):
            g[h].wait()
            if h + 4 < K:
                if h >= 2:
                    st[h - 2].wait()
                g[h + 4] = _gather(h + 4, (h + 4) % 6, row)
            st[h] = _store(h, h % 6, row)
        st[K - 2].wait()
        st[K - 1].wait()
        return carry

    lax.fori_loop(0, NUM_BLOCKS, body, 0)


def kernel(token_ids, weight):
    idx = token_ids.reshape(B_TOTAL // CHUNK, CHUNK).astype(jnp.int32)
    wp = jnp.pad(weight, ((0, 0), (0, D_PAD - D_MODEL)))
    out = _gather_kernel(idx, wp)
    return out[:, :D_MODEL].reshape(token_ids.shape + (D_MODEL,))


# speedup vs baseline: 1.0020x; 1.0001x over previous
import functools

import jax
import jax.numpy as jnp
from jax import lax
from jax.experimental import pallas as pl
from jax.experimental.pallas import tpu as pltpu
from jax.experimental.pallas import tpu_sc as plsc

D_MODEL = 64
D_PAD = 128
B_TOTAL = 16384 * 20
NUM_WORKERS = 32
CHUNK = 128
K = 8
ROWS_PER_W = B_TOTAL // (NUM_WORKERS * CHUNK)
NUM_BLOCKS = ROWS_PER_W // K

_mesh = plsc.VectorSubcoreMesh(core_axis_name="c", subcore_axis_name="s")


@functools.partial(
    pl.kernel,
    mesh=_mesh,
    out_type=jax.ShapeDtypeStruct((B_TOTAL, D_PAD), jnp.float32),
    scratch_types=[
        pltpu.VMEM((K, CHUNK), jnp.int32),
        pltpu.VMEM((6, CHUNK, D_PAD), jnp.float32),
        pltpu.SemaphoreType.DMA((6,)),
        pltpu.SemaphoreType.DMA((6,)),
    ],
)
def _gather_kernel(idx_hbm, table_hbm, out_hbm, idx_v, rows_v, sem_g, sem_s):
    wid = lax.axis_index("s") * 2 + lax.axis_index("c")
    base_row = wid * ROWS_PER_W

    def _gather(h, buf, row):
        return pltpu.async_copy(table_hbm.at[idx_v.at[h]], rows_v.at[buf],
                                sem_g.at[buf])

    def _store(h, buf, row):
        return pltpu.async_copy(
            rows_v.at[buf],
            out_hbm.at[pl.ds((row + h) * CHUNK, CHUNK)], sem_s.at[buf])

    def body(blk, carry):
        row = base_row + blk * K
        pltpu.sync_copy(idx_hbm.at[pl.ds(row, K)], idx_v)
        g = [None] * K
        st = [None] * K
        for h in range(4):
            g[h] = _gather(h, h % 6, row)
        for h in range(K):
            g[h].wait()
            if h + 4 < K:
                if h >= 2:
                    st[h - 2].wait()
                g[h + 4] = _gather(h + 4, (h + 4) % 6, row)
            st[h] = _store(h, h % 6, row)
        for h in range(max(0, K - 6), K):
            st[h].wait()
        return carry

    lax.fori_loop(0, NUM_BLOCKS, body, 0)


def kernel(token_ids, weight):
    idx = token_ids.reshape(B_TOTAL // CHUNK, CHUNK).astype(jnp.int32)
    wp = jnp.pad(weight, ((0, 0), (0, D_PAD - D_MODEL)))
    out = _gather_kernel(idx, wp)
    return out[:, :D_MODEL].reshape(token_ids.shape + (D_MODEL,))
